# baseline (speedup 1.0000x reference)
import jax
import jax.numpy as jnp
from jax.experimental import pallas as pl

N = 48000
G = 32
H = 36
OUT = 18
L = 10
T = 3
MAXLEN = 40
NCE = 0.25


def _seg_softmax(logits, seg, num):
    m = jax.ops.segment_max(logits, seg, num_segments=num)
    m = jnp.where(jnp.isfinite(m), m, 0.0)
    e = jnp.exp(logits - m[seg])
    z = jax.ops.segment_sum(e, seg, num_segments=num)
    return e / (z[seg] + 1e-16)


def _gru(xi, h, Wih, Whh, bih, bhh):
    gi = xi @ Wih + bih
    gh = h @ Whh + bhh
    ir, iz, inn = jnp.split(gi, 3, axis=-1)
    hr, hz, hn = jnp.split(gh, 3, axis=-1)
    r = jax.nn.sigmoid(ir + hr)
    z = jax.nn.sigmoid(iz + hz)
    n = jnp.tanh(inn + r * hn)
    return (1.0 - z) * n + z * h


def _final_stage_kernel(g_ref, W2_ref, charge_ref, out_ref):
    emb = g_ref[...] @ W2_ref[...]
    pad = jnp.zeros((MAXLEN - G, OUT), dtype=emb.dtype)
    emb = jnp.concatenate([emb, pad], axis=0)
    col = jax.lax.broadcasted_iota(jnp.int32, (MAXLEN, OUT), 1)
    row = jax.lax.broadcasted_iota(jnp.int32, (MAXLEN, OUT), 0)
    is_last = row == MAXLEN - 1
    emb = jnp.where(is_last & (col == charge_ref[0]), 1.0, emb)
    emb = jnp.where(is_last & (col == OUT - 1), NCE, emb)
    out_ref[...] = emb


def kernel(x, edge_index, edge_attr, graph_ids, charge, W1, We, att_e, att_d, Wm0, att_src, att_dst, Wm, gWih, gWhh, gbih, gbhh, w_read, W_read, mWih, mWhh, mbih, mbhh, W2):
    src = edge_index[0]
    dst = edge_index[1]
    h = jax.nn.leaky_relu(x @ W1)
    me = jax.nn.leaky_relu(jnp.concatenate([h[src], edge_attr], axis=1) @ We)
    logits = jax.nn.leaky_relu(me @ att_e + (h @ att_d)[dst])
    alpha = _seg_softmax(logits, dst, N)
    agg = jax.ops.segment_sum(alpha[:, None] * (h[src] @ Wm0), dst, num_segments=N)
    h = jax.nn.elu(_gru(agg, h, gWih, gWhh, gbih, gbhh))
    for l in range(L - 1):
        logits = jax.nn.leaky_relu((h @ att_src[l])[src] + (h @ att_dst[l])[dst])
        alpha = _seg_softmax(logits, dst, N)
        agg = jax.ops.segment_sum(alpha[:, None] * (h[src] @ Wm[l]), dst, num_segments=N)
        h = jax.nn.elu(_gru(agg, h, gWih, gWhh, gbih, gbhh))
    g = jax.ops.segment_sum(h, graph_ids, num_segments=G)
    for t in range(T):
        rl = jax.nn.leaky_relu(jnp.concatenate([g[graph_ids], h], axis=1) @ w_read)
        beta = _seg_softmax(rl, graph_ids, G)
        msg = jax.ops.segment_sum(beta[:, None] * (h @ W_read), graph_ids, num_segments=G)
        g = _gru(jax.nn.elu(msg), g, mWih, mWhh, mbih, mbhh)
    charge_arr = jnp.asarray(charge, dtype=jnp.int32).reshape((1,))
    out = pl.pallas_call(
        _final_stage_kernel,
        out_shape=jax.ShapeDtypeStruct((MAXLEN, OUT), jnp.float32),
    )(g, W2, charge_arr)
    return out


# speedup vs baseline: 15.0814x; 15.0814x over previous
import functools

import jax
import jax.numpy as jnp
from jax import lax
from jax.experimental import pallas as pl
from jax.experimental.pallas import tpu as pltpu
from jax.experimental.pallas import tpu_sc as plsc

N = 48000
E = 768000
G = 32
H = 36
OUT = 18
L = 10
T = 3
MAXLEN = 40
NCE = 0.25

W40 = 40
C = 1600
EPW = E // 32
NCH = EPW // C
NPT = N // 16
NBLK = 1920
EBLK = 3200


def _sc_gather_body(table_hbm, src_hbm, out_hbm, idx_v, rows_v, sem):
    c = lax.axis_index("c")
    s = lax.axis_index("s")
    base = (c * 16 + s) * EPW

    def chunk(i, carry):
        off = base + i * C
        pltpu.sync_copy(src_hbm.at[pl.ds(off, C)], idx_v)
        pltpu.async_copy(table_hbm.at[idx_v], rows_v, sem).wait()
        pltpu.sync_copy(rows_v, out_hbm.at[pl.ds(off, C)])
        return carry

    lax.fori_loop(0, NCH, chunk, 0)


def _sc_edge_body(u_hbm, dst_hbm, adv_hbm, zeros_hbm, e_hbm, zp_hbm,
                  adv_v, dst_v, u_v, e_v, z_sh):
    c = lax.axis_index("c")
    s = lax.axis_index("s")
    base = (c * 16 + s) * EPW
    pltpu.sync_copy(adv_hbm, adv_v)
    pltpu.sync_copy(zeros_hbm.at[pl.ds(s * NPT, NPT)], z_sh.at[pl.ds(s * NPT, NPT)])
    plsc.subcore_barrier()

    def chunk(i, carry):
        off = base + i * C
        pltpu.sync_copy(dst_hbm.at[pl.ds(off, C)], dst_v)
        pltpu.sync_copy(u_hbm.at[pl.ds(off, C)], u_v)

        def vec(j, c2):
            dv = dst_v[pl.ds(j * 16, 16)]
            av = plsc.load_gather(adv_v, [dv])
            t = u_v[pl.ds(j * 16, 16)] + av
            t = jnp.where(t >= 0.0, t, 0.01 * t)
            e_v[pl.ds(j * 16, 16)] = jnp.exp(t)
            return c2

        lax.fori_loop(0, C // 16, vec, 0)
        pltpu.sync_copy(e_v, e_hbm.at[pl.ds(off, C)])
        pltpu.sync_copy(e_v, z_sh.at[dst_v], add=True)
        return carry

    lax.fori_loop(0, NCH, chunk, 0)
    plsc.subcore_barrier()
    pltpu.sync_copy(z_sh.at[pl.ds(s * NPT, NPT)], zp_hbm.at[c, pl.ds(s * NPT, NPT)])


HALFN = N // 2
AGGROWS = HALFN + 16
ZR = AGGROWS // 16
WBR = HALFN // 16
EPT2 = E // 16


def _sc_scatter_body(dst_hbm, rows_hbm, zeros_hbm, agg_hbm, dst_v, idx_v, rows_v, agg_sh):
    c = lax.axis_index("c")
    s = lax.axis_index("s")
    base = s * EPT2
    pltpu.sync_copy(zeros_hbm, agg_sh.at[pl.ds(s * ZR, ZR)])
    plsc.subcore_barrier()
    half0 = c * HALFN

    def chunk(i, carry):
        off = base + i * C
        pltpu.sync_copy(dst_hbm.at[pl.ds(off, C)], dst_v)
        pltpu.sync_copy(rows_hbm.at[pl.ds(off, C)], rows_v)

        def vec(j, c2):
            dv = dst_v[pl.ds(j * 16, 16)] - half0
            valid = (dv >= 0) & (dv < HALFN)
            idx_v[pl.ds(j * 16, 16)] = jnp.where(valid, dv, HALFN)
            return c2

        lax.fori_loop(0, C // 16, vec, 0)
        pltpu.sync_copy(rows_v, agg_sh.at[idx_v], add=True)
        return carry

    lax.fori_loop(0, EPT2 // C, chunk, 0)
    plsc.subcore_barrier()
    pltpu.sync_copy(agg_sh.at[pl.ds(s * WBR, WBR)],
                    agg_hbm.at[pl.ds(c * HALFN + s * WBR, WBR)])


@functools.cache
def _sc_kernels():
    mesh = plsc.VectorSubcoreMesh(core_axis_name="c", subcore_axis_name="s")
    gather = pl.kernel(
        _sc_gather_body,
        out_type=jax.ShapeDtypeStruct((E, W40), jnp.float32),
        mesh=mesh,
        compiler_params=pltpu.CompilerParams(use_tc_tiling_on_sc=False, needs_layout_passes=False),
        scratch_types=[
            pltpu.VMEM((C,), jnp.int32),
            pltpu.VMEM((C, W40), jnp.float32),
            pltpu.SemaphoreType.DMA,
        ],
    )
    edge = pl.kernel(
        _sc_edge_body,
        out_type=(
            jax.ShapeDtypeStruct((E,), jnp.float32),
            jax.ShapeDtypeStruct((2, N), jnp.float32),
        ),
        mesh=mesh,
        compiler_params=pltpu.CompilerParams(use_tc_tiling_on_sc=False, needs_layout_passes=False),
        scratch_types=[
            pltpu.VMEM((N,), jnp.float32),
            pltpu.VMEM((C,), jnp.int32),
            pltpu.VMEM((C,), jnp.float32),
            pltpu.VMEM((C,), jnp.float32),
            pltpu.VMEM_SHARED((N,), jnp.float32),
        ],
    )
    scatter = pl.kernel(
        _sc_scatter_body,
        out_type=jax.ShapeDtypeStruct((N, W40), jnp.float32),
        mesh=mesh,
        compiler_params=pltpu.CompilerParams(use_tc_tiling_on_sc=False, needs_layout_passes=False),
        scratch_types=[
            pltpu.VMEM((C,), jnp.int32),
            pltpu.VMEM((C,), jnp.int32),
            pltpu.VMEM((C, W40), jnp.float32),
            pltpu.VMEM_SHARED((AGGROWS, W40), jnp.float32),
        ],
    )
    return gather, edge, scatter


def _sc_gather(table, src):
    return _sc_kernels()[0](table, src)


def _sc_edge(u, dst, advec, zeros_n):
    return _sc_kernels()[1](u, dst, advec, zeros_n)


def _sc_scatter(dst, scaled, zeros_rows):
    return _sc_kernels()[2](dst, scaled, zeros_rows)


def _mm_body(x_ref, w_ref, o_ref, *, act):
    y = jnp.dot(x_ref[...], w_ref[...], preferred_element_type=jnp.float32)
    if act == "leaky":
        y = jnp.where(y >= 0.0, y, 0.01 * y)
    o_ref[...] = y


def _mm(x, w, act=None, blk=NBLK):
    n, k = x.shape
    m = w.shape[1]
    return pl.pallas_call(
        functools.partial(_mm_body, act=act),
        grid=(n // blk,),
        in_specs=[
            pl.BlockSpec((blk, k), lambda i: (i, 0)),
            pl.BlockSpec((k, m), lambda i: (0, 0)),
        ],
        out_specs=pl.BlockSpec((blk, m), lambda i: (i, 0)),
        out_shape=jax.ShapeDtypeStruct((n, m), jnp.float32),
    )(x, w)


def _scale_body(r_ref, e_ref, o_ref):
    o_ref[...] = r_ref[...] * e_ref[...]


def _scale_rows(rows, e):
    return pl.pallas_call(
        _scale_body,
        grid=(E // EBLK,),
        in_specs=[
            pl.BlockSpec((EBLK, W40), lambda i: (i, 0)),
            pl.BlockSpec((EBLK, 1), lambda i: (i, 0)),
        ],
        out_specs=pl.BlockSpec((EBLK, W40), lambda i: (i, 0)),
        out_shape=jax.ShapeDtypeStruct((E, W40), jnp.float32),
    )(rows, e)


def _edge0_body(rw_ref, ea_ref, wee_ref, atte_ref, o_ref):
    me = rw_ref[...][:, :H] + jnp.dot(ea_ref[...], wee_ref[...],
                                      preferred_element_type=jnp.float32)
    me = jnp.where(me >= 0.0, me, 0.01 * me)
    o_ref[...] = jnp.dot(me, atte_ref[...], preferred_element_type=jnp.float32)


def _edge0(rowsW, edge_attr, We_e, att_e):
    return pl.pallas_call(
        _edge0_body,
        grid=(E // EBLK,),
        in_specs=[
            pl.BlockSpec((EBLK, W40), lambda i: (i, 0)),
            pl.BlockSpec((EBLK, 9), lambda i: (i, 0)),
            pl.BlockSpec((9, H), lambda i: (0, 0)),
            pl.BlockSpec((H, 1), lambda i: (0, 0)),
        ],
        out_specs=pl.BlockSpec((EBLK, 1), lambda i: (i, 0)),
        out_shape=jax.ShapeDtypeStruct((E, 1), jnp.float32),
    )(rowsW, edge_attr, We_e, att_e)


def _gru_math(xi, h, ws):
    wir, wiz, win, whr, whz, whn, bir, biz, bin_, bhr, bhz, bhn = ws
    i_r = jnp.dot(xi, wir, preferred_element_type=jnp.float32) + bir
    i_z = jnp.dot(xi, wiz, preferred_element_type=jnp.float32) + biz
    i_n = jnp.dot(xi, win, preferred_element_type=jnp.float32) + bin_
    h_r = jnp.dot(h, whr, preferred_element_type=jnp.float32) + bhr
    h_z = jnp.dot(h, whz, preferred_element_type=jnp.float32) + bhz
    h_n = jnp.dot(h, whn, preferred_element_type=jnp.float32) + bhn
    r = jax.nn.sigmoid(i_r + h_r)
    z = jax.nn.sigmoid(i_z + h_z)
    n = jnp.tanh(i_n + r * h_n)
    return (1.0 - z) * n + z * h


def _gru_node_body(a_ref, zs_ref, h_ref, *rest):
    wrefs = rest[:-1]
    o_ref = rest[-1]
    zs = zs_ref[...]
    r_ = 1.0 / (zs[:, 0:1] + zs[:, 1:2] + 1e-16)
    agg = a_ref[...][:, :H] * r_
    ws = tuple(w[...] for w in wrefs)
    v = _gru_math(agg, h_ref[...], ws)
    o_ref[...] = jnp.where(v > 0.0, v, jnp.exp(v) - 1.0)


def _gru_node(a, zs, h, ws):
    wspecs = [pl.BlockSpec(w.shape, lambda i: tuple(0 for _ in w.shape)) for w in ws]
    return pl.pallas_call(
        _gru_node_body,
        grid=(N // NBLK,),
        in_specs=[
            pl.BlockSpec((NBLK, W40), lambda i: (i, 0)),
            pl.BlockSpec((NBLK, 2), lambda i: (i, 0)),
            pl.BlockSpec((NBLK, H), lambda i: (i, 0)),
        ] + wspecs,
        out_specs=pl.BlockSpec((NBLK, H), lambda i: (i, 0)),
        out_shape=jax.ShapeDtypeStruct((N, H), jnp.float32),
    )(a, zs, h, *ws)


def _hread_body(h_ref, wr_ref, w2_ref, o1_ref, o2_ref):
    h = h_ref[...]
    o1_ref[...] = jnp.dot(h, wr_ref[...], preferred_element_type=jnp.float32)
    o2_ref[...] = jnp.dot(h, w2_ref[...], preferred_element_type=jnp.float32)


def _hread(h, W_read, w2col):
    return pl.pallas_call(
        _hread_body,
        grid=(N // NBLK,),
        in_specs=[
            pl.BlockSpec((NBLK, H), lambda i: (i, 0)),
            pl.BlockSpec((H, H), lambda i: (0, 0)),
            pl.BlockSpec((H, 1), lambda i: (0, 0)),
        ],
        out_specs=[
            pl.BlockSpec((NBLK, H), lambda i: (i, 0)),
            pl.BlockSpec((NBLK, 1), lambda i: (i, 0)),
        ],
        out_shape=[
            jax.ShapeDtypeStruct((N, H), jnp.float32),
            jax.ShapeDtypeStruct((N, 1), jnp.float32),
        ],
    )(h, W_read, w2col)


def _seg0_body(gidr_ref, h_ref, o_ref):
    i = pl.program_id(0)
    oh_t = (gidr_ref[...] == lax.broadcasted_iota(jnp.int32, (G, NBLK), 0)
            ).astype(jnp.float32)
    part = jnp.dot(oh_t, h_ref[...], preferred_element_type=jnp.float32)

    @pl.when(i == 0)
    def _():
        o_ref[...] = jnp.zeros_like(o_ref)

    o_ref[...] += part


def _seg0(gids_row, h):
    return pl.pallas_call(
        _seg0_body,
        grid=(N // NBLK,),
        in_specs=[
            pl.BlockSpec((1, NBLK), lambda i: (0, i)),
            pl.BlockSpec((NBLK, H), lambda i: (i, 0)),
        ],
        out_specs=pl.BlockSpec((G, H), lambda i: (0, 0)),
        out_shape=jax.ShapeDtypeStruct((G, H), jnp.float32),
    )(gids_row, h)


def _tstep_body(g_ref, gidc_ref, gidr_ref, c2_ref, hwr_ref, w1_ref,
                zg_ref, msg_ref):
    i = pl.program_id(0)
    gw = jnp.dot(g_ref[...], w1_ref[...], preferred_element_type=jnp.float32)
    oh = (gidc_ref[...] == lax.broadcasted_iota(jnp.int32, (NBLK, G), 1)
          ).astype(jnp.float32)
    oh_t = (gidr_ref[...] == lax.broadcasted_iota(jnp.int32, (G, NBLK), 0)
            ).astype(jnp.float32)
    rl = jnp.dot(oh, gw, preferred_element_type=jnp.float32) + c2_ref[...]
    rl = jnp.where(rl >= 0.0, rl, 0.01 * rl)
    q = jnp.exp(rl)
    zg_part = jnp.dot(oh_t, q, preferred_element_type=jnp.float32)
    msg_part = jnp.dot(oh_t, q * hwr_ref[...], preferred_element_type=jnp.float32)

    @pl.when(i == 0)
    def _():
        zg_ref[...] = jnp.zeros_like(zg_ref)
        msg_ref[...] = jnp.zeros_like(msg_ref)

    zg_ref[...] += zg_part
    msg_ref[...] += msg_part


def _tstep(g, gids_col, gids_row, c2, hWr, w1col):
    return pl.pallas_call(
        _tstep_body,
        grid=(N // NBLK,),
        in_specs=[
            pl.BlockSpec((G, H), lambda i: (0, 0)),
            pl.BlockSpec((NBLK, 1), lambda i: (i, 0)),
            pl.BlockSpec((1, NBLK), lambda i: (0, i)),
            pl.BlockSpec((NBLK, 1), lambda i: (i, 0)),
            pl.BlockSpec((NBLK, H), lambda i: (i, 0)),
            pl.BlockSpec((H, 1), lambda i: (0, 0)),
        ],
        out_specs=[
            pl.BlockSpec((G, 1), lambda i: (0, 0)),
            pl.BlockSpec((G, H), lambda i: (0, 0)),
        ],
        out_shape=[
            jax.ShapeDtypeStruct((G, 1), jnp.float32),
            jax.ShapeDtypeStruct((G, H), jnp.float32),
        ],
    )(g, gids_col, gids_row, c2, hWr, w1col)


def _tgru_body(msg_ref, zg_ref, g_ref, *rest):
    wrefs = rest[:-1]
    o_ref = rest[-1]
    msg = msg_ref[...] * (1.0 / (zg_ref[...] + 1e-16))
    xi = jnp.where(msg > 0.0, msg, jnp.exp(msg) - 1.0)
    ws = tuple(w[...] for w in wrefs)
    o_ref[...] = _gru_math(xi, g_ref[...], ws)


def _tgru(msgp, zg, g, ws):
    wspecs = [pl.BlockSpec(w.shape, lambda i: tuple(0 for _ in w.shape)) for w in ws]
    return pl.pallas_call(
        _tgru_body,
        grid=(1,),
        in_specs=[
            pl.BlockSpec((G, H), lambda i: (0, 0)),
            pl.BlockSpec((G, 1), lambda i: (0, 0)),
            pl.BlockSpec((G, H), lambda i: (0, 0)),
        ] + wspecs,
        out_specs=pl.BlockSpec((G, H), lambda i: (0, 0)),
        out_shape=jax.ShapeDtypeStruct((G, H), jnp.float32),
    )(msgp, zg, g, *ws)


def _final_body(g_ref, W2_ref, charge_ref, out_ref):
    emb = jnp.dot(g_ref[...], W2_ref[...], preferred_element_type=jnp.float32)
    emb = jnp.concatenate([emb, jnp.zeros((MAXLEN - G, OUT), jnp.float32)], axis=0)
    col = lax.broadcasted_iota(jnp.int32, (MAXLEN, OUT), 1)
    row = lax.broadcasted_iota(jnp.int32, (MAXLEN, OUT), 0)
    is_last = row == MAXLEN - 1
    emb = jnp.where(is_last & (col == charge_ref[0]), 1.0, emb)
    emb = jnp.where(is_last & (col == OUT - 1), NCE, emb)
    out_ref[...] = emb


def _split3(Wih, Whh, bih, bhh):
    ws = [Wih[:, :H], Wih[:, H:2 * H], Wih[:, 2 * H:],
          Whh[:, :H], Whh[:, H:2 * H], Whh[:, 2 * H:],
          bih[:H].reshape(1, H), bih[H:2 * H].reshape(1, H), bih[2 * H:].reshape(1, H),
          bhh[:H].reshape(1, H), bhh[H:2 * H].reshape(1, H), bhh[2 * H:].reshape(1, H)]
    return tuple(ws)


def kernel(x, edge_index, edge_attr, graph_ids, charge, W1, We, att_e, att_d,
           Wm0, att_src, att_dst, Wm, gWih, gWhh, gbih, gbhh, w_read, W_read,
           mWih, mWhh, mbih, mbhh, W2):
    src = edge_index[0]
    dst = edge_index[1]
    zeros_n = jnp.zeros((N,), jnp.float32)
    zeros_rows = jnp.zeros((ZR, W40), jnp.float32)
    gws = _split3(gWih, gWhh, gbih, gbhh)
    mws = _split3(mWih, mWhh, mbih, mbhh)
    z36 = jnp.zeros((H, 1), jnp.float32)

    h = _mm(x, W1, act="leaky")

    def sparse_layer(h, Wtab, u, advec):
        table = _mm(h, Wtab)
        rows = _sc_gather(table, src)
        e, zp = _sc_edge(u.reshape(E), dst, advec, zeros_n)
        scaled = _scale_rows(rows, e.reshape(E, 1))
        agg = _sc_scatter(dst, scaled, zeros_rows)
        return _gru_node(agg, zp.T, h, gws), rows

    Wtab0 = jnp.concatenate([We[:H], att_d.reshape(H, 1), z36, z36, z36], axis=1)
    table0 = _mm(h, Wtab0)
    rowsW = _sc_gather(table0, src)
    u0 = _edge0(rowsW, edge_attr, We[H:], att_e.reshape(H, 1))
    Wm0t = jnp.concatenate([Wm0, z36, z36, z36, z36], axis=1)
    advec0 = table0[:, 36]
    h, _ = sparse_layer(h, Wm0t, u0, advec0)

    for l in range(L - 1):
        Wtab = jnp.concatenate(
            [Wm[l], att_src[l].reshape(H, 1), att_dst[l].reshape(H, 1), z36, z36],
            axis=1)
        table = _mm(h, Wtab)
        rows = _sc_gather(table, src)
        u = rows[:, 36]
        advec = table[:, 37]
        e, zp = _sc_edge(u, dst, advec, zeros_n)
        scaled = _scale_rows(rows, e.reshape(E, 1))
        agg = _sc_scatter(dst, scaled, zeros_rows)
        h = _gru_node(agg, zp.T, h, gws)

    gids_row = graph_ids.reshape(1, N)
    gids_col = graph_ids.reshape(N, 1)
    g = _seg0(gids_row, h)
    hWr, c2 = _hread(h, W_read, w_read[H:].reshape(H, 1))
    w1col = w_read[:H].reshape(H, 1)
    for t in range(T):
        zg, msgp = _tstep(g, gids_col, gids_row, c2, hWr, w1col)
        g = _tgru(msgp, zg, g, mws)

    charge_arr = jnp.asarray(charge, dtype=jnp.int32).reshape((1,))
    out = pl.pallas_call(
        _final_body,
        out_shape=jax.ShapeDtypeStruct((MAXLEN, OUT), jnp.float32),
    )(g, W2, charge_arr)
    return out


# speedup vs baseline: 16.5921x; 1.1002x over previous
import functools

import jax
import jax.numpy as jnp
from jax import lax
from jax.experimental import pallas as pl
from jax.experimental.pallas import tpu as pltpu
from jax.experimental.pallas import tpu_sc as plsc

N = 48000
E = 768000
G = 32
H = 36
OUT = 18
L = 10
T = 3
MAXLEN = 40
NCE = 0.25

W40 = 40
C = 1600
EPW = E // 32
NCH = EPW // C
NPT = N // 16
NBLK = 1920
EBLK = 3200


def _sc_gather_body(table_hbm, src_hbm, asvec_hbm, out_hbm, u_hbm,
                    as_v, idx_v, rows_v, u_v, sem):
    c = lax.axis_index("c")
    s = lax.axis_index("s")
    base = (c * 16 + s) * EPW
    pltpu.sync_copy(asvec_hbm, as_v)

    def chunk(i, carry):
        off = base + i * C
        pltpu.sync_copy(src_hbm.at[pl.ds(off, C)], idx_v)
        desc = pltpu.async_copy(table_hbm.at[idx_v], rows_v, sem)

        def vec(j, c2):
            u_v[pl.ds(j * 16, 16)] = plsc.load_gather(as_v, [idx_v[pl.ds(j * 16, 16)]])
            return c2

        lax.fori_loop(0, C // 16, vec, 0)
        pltpu.sync_copy(u_v, u_hbm.at[pl.ds(off, C)])
        desc.wait()
        pltpu.sync_copy(rows_v, out_hbm.at[pl.ds(off, C)])
        return carry

    lax.fori_loop(0, NCH, chunk, 0)


def _sc_edge_body(u_hbm, dst_hbm, adv_hbm, zeros_hbm, e_hbm, zp_hbm,
                  adv_v, z_v, dst_v, u_v, e_v):
    c = lax.axis_index("c")
    s = lax.axis_index("s")
    wid = c * 16 + s
    base = wid * EPW
    pltpu.sync_copy(adv_hbm, adv_v)
    pltpu.sync_copy(zeros_hbm, z_v)

    def chunk(i, carry):
        off = base + i * C
        pltpu.sync_copy(dst_hbm.at[pl.ds(off, C)], dst_v)
        pltpu.sync_copy(u_hbm.at[pl.ds(off, C)], u_v)

        def vec(j, c2):
            dv = dst_v[pl.ds(j * 16, 16)]
            av = plsc.load_gather(adv_v, [dv])
            t = u_v[pl.ds(j * 16, 16)] + av
            t = jnp.where(t >= 0.0, t, 0.01 * t)
            ev = jnp.exp(t)
            e_v[pl.ds(j * 16, 16)] = ev
            plsc.addupdate_scatter(z_v, [dv], ev)
            return c2

        lax.fori_loop(0, C // 16, vec, 0)
        pltpu.sync_copy(e_v, e_hbm.at[pl.ds(off, C)])
        return carry

    lax.fori_loop(0, NCH, chunk, 0)
    pltpu.sync_copy(z_v, zp_hbm.at[wid])


HALFN = N // 2
AGGROWS = HALFN + 16
ZR = AGGROWS // 16
WBR = HALFN // 16
EPT2 = E // 16


def _sc_scatter_body(dst_hbm, rows_hbm, zeros_hbm, agg_hbm, dst_v, idx_v, rows_v, agg_sh):
    c = lax.axis_index("c")
    s = lax.axis_index("s")
    base = s * EPT2
    pltpu.sync_copy(zeros_hbm, agg_sh.at[pl.ds(s * ZR, ZR)])
    plsc.subcore_barrier()
    half0 = c * HALFN

    def chunk(i, carry):
        off = base + i * C
        pltpu.sync_copy(dst_hbm.at[pl.ds(off, C)], dst_v)
        pltpu.sync_copy(rows_hbm.at[pl.ds(off, C)], rows_v)

        def vec(j, c2):
            dv = dst_v[pl.ds(j * 16, 16)] - half0
            valid = (dv >= 0) & (dv < HALFN)
            idx_v[pl.ds(j * 16, 16)] = jnp.where(valid, dv, HALFN)
            return c2

        lax.fori_loop(0, C // 16, vec, 0)
        pltpu.sync_copy(rows_v, agg_sh.at[idx_v], add=True)
        return carry

    lax.fori_loop(0, EPT2 // C, chunk, 0)
    plsc.subcore_barrier()
    pltpu.sync_copy(agg_sh.at[pl.ds(s * WBR, WBR)],
                    agg_hbm.at[pl.ds(c * HALFN + s * WBR, WBR)])


@functools.cache
def _sc_kernels():
    mesh = plsc.VectorSubcoreMesh(core_axis_name="c", subcore_axis_name="s")
    gather = pl.kernel(
        _sc_gather_body,
        out_type=(
            jax.ShapeDtypeStruct((E, W40), jnp.float32),
            jax.ShapeDtypeStruct((E,), jnp.float32),
        ),
        mesh=mesh,
        compiler_params=pltpu.CompilerParams(use_tc_tiling_on_sc=False, needs_layout_passes=False),
        scratch_types=[
            pltpu.VMEM((N,), jnp.float32),
            pltpu.VMEM((C,), jnp.int32),
            pltpu.VMEM((C, W40), jnp.float32),
            pltpu.VMEM((C,), jnp.float32),
            pltpu.SemaphoreType.DMA,
        ],
    )
    edge = pl.kernel(
        _sc_edge_body,
        out_type=(
            jax.ShapeDtypeStruct((E,), jnp.float32),
            jax.ShapeDtypeStruct((32, N), jnp.float32),
        ),
        mesh=mesh,
        compiler_params=pltpu.CompilerParams(use_tc_tiling_on_sc=False, needs_layout_passes=False),
        scratch_types=[
            pltpu.VMEM((N,), jnp.float32),
            pltpu.VMEM((N,), jnp.float32),
            pltpu.VMEM((C,), jnp.int32),
            pltpu.VMEM((C,), jnp.float32),
            pltpu.VMEM((C,), jnp.float32),
        ],
    )
    scatter = pl.kernel(
        _sc_scatter_body,
        out_type=jax.ShapeDtypeStruct((N, W40), jnp.float32),
        mesh=mesh,
        compiler_params=pltpu.CompilerParams(use_tc_tiling_on_sc=False, needs_layout_passes=False),
        scratch_types=[
            pltpu.VMEM((C,), jnp.int32),
            pltpu.VMEM((C,), jnp.int32),
            pltpu.VMEM((C, W40), jnp.float32),
            pltpu.VMEM_SHARED((AGGROWS, W40), jnp.float32),
        ],
    )
    return gather, edge, scatter


def _sc_gather(table, src, asvec):
    return _sc_kernels()[0](table, src, asvec)


def _sc_edge(u, dst, advec, zeros_n):
    return _sc_kernels()[1](u, dst, advec, zeros_n)


def _sc_scatter(dst, scaled, zeros_rows):
    return _sc_kernels()[2](dst, scaled, zeros_rows)


def _mm_body(x_ref, w_ref, o_ref, *, act):
    y = jnp.dot(x_ref[...], w_ref[...], preferred_element_type=jnp.float32)
    if act == "leaky":
        y = jnp.where(y >= 0.0, y, 0.01 * y)
    o_ref[...] = y


def _mm(x, w, act=None, blk=NBLK):
    n, k = x.shape
    m = w.shape[1]
    return pl.pallas_call(
        functools.partial(_mm_body, act=act),
        grid=(n // blk,),
        in_specs=[
            pl.BlockSpec((blk, k), lambda i: (i, 0)),
            pl.BlockSpec((k, m), lambda i: (0, 0)),
        ],
        out_specs=pl.BlockSpec((blk, m), lambda i: (i, 0)),
        out_shape=jax.ShapeDtypeStruct((n, m), jnp.float32),
    )(x, w)


def _scale_body(r_ref, e_ref, o_ref):
    o_ref[...] = r_ref[...] * e_ref[...]


def _scale_rows(rows, e):
    return pl.pallas_call(
        _scale_body,
        grid=(E // EBLK,),
        in_specs=[
            pl.BlockSpec((EBLK, W40), lambda i: (i, 0)),
            pl.BlockSpec((EBLK, 1), lambda i: (i, 0)),
        ],
        out_specs=pl.BlockSpec((EBLK, W40), lambda i: (i, 0)),
        out_shape=jax.ShapeDtypeStruct((E, W40), jnp.float32),
    )(rows, e)


def _edge0_body(rw_ref, ea_ref, wee_ref, atte_ref, o_ref):
    me = rw_ref[...][:, :H] + jnp.dot(ea_ref[...], wee_ref[...],
                                      preferred_element_type=jnp.float32)
    me = jnp.where(me >= 0.0, me, 0.01 * me)
    o_ref[...] = jnp.dot(me, atte_ref[...], preferred_element_type=jnp.float32)


def _edge0(rowsW, edge_attr, We_e, att_e):
    return pl.pallas_call(
        _edge0_body,
        grid=(E // EBLK,),
        in_specs=[
            pl.BlockSpec((EBLK, W40), lambda i: (i, 0)),
            pl.BlockSpec((EBLK, 9), lambda i: (i, 0)),
            pl.BlockSpec((9, H), lambda i: (0, 0)),
            pl.BlockSpec((H, 1), lambda i: (0, 0)),
        ],
        out_specs=pl.BlockSpec((EBLK, 1), lambda i: (i, 0)),
        out_shape=jax.ShapeDtypeStruct((E, 1), jnp.float32),
    )(rowsW, edge_attr, We_e, att_e)


def _gru_math(xi, h, ws):
    wir, wiz, win, whr, whz, whn, bir, biz, bin_, bhr, bhz, bhn = ws
    i_r = jnp.dot(xi, wir, preferred_element_type=jnp.float32) + bir
    i_z = jnp.dot(xi, wiz, preferred_element_type=jnp.float32) + biz
    i_n = jnp.dot(xi, win, preferred_element_type=jnp.float32) + bin_
    h_r = jnp.dot(h, whr, preferred_element_type=jnp.float32) + bhr
    h_z = jnp.dot(h, whz, preferred_element_type=jnp.float32) + bhz
    h_n = jnp.dot(h, whn, preferred_element_type=jnp.float32) + bhn
    r = jax.nn.sigmoid(i_r + h_r)
    z = jax.nn.sigmoid(i_z + h_z)
    n = jnp.tanh(i_n + r * h_n)
    return (1.0 - z) * n + z * h


def _gru_node_body(a_ref, zs_ref, h_ref, *rest):
    wrefs = rest[:-1]
    o_ref = rest[-1]
    r_ = 1.0 / (jnp.sum(zs_ref[...], axis=1, keepdims=True) + 1e-16)
    agg = a_ref[...][:, :H] * r_
    ws = tuple(w[...] for w in wrefs)
    v = _gru_math(agg, h_ref[...], ws)
    o_ref[...] = jnp.where(v > 0.0, v, jnp.exp(v) - 1.0)


def _gru_node(a, zs, h, ws):
    wspecs = [pl.BlockSpec(w.shape, lambda i: tuple(0 for _ in w.shape)) for w in ws]
    return pl.pallas_call(
        _gru_node_body,
        grid=(N // NBLK,),
        in_specs=[
            pl.BlockSpec((NBLK, W40), lambda i: (i, 0)),
            pl.BlockSpec((NBLK, 32), lambda i: (i, 0)),
            pl.BlockSpec((NBLK, H), lambda i: (i, 0)),
        ] + wspecs,
        out_specs=pl.BlockSpec((NBLK, H), lambda i: (i, 0)),
        out_shape=jax.ShapeDtypeStruct((N, H), jnp.float32),
    )(a, zs, h, *ws)


def _hread_body(h_ref, wr_ref, w2_ref, o1_ref, o2_ref):
    h = h_ref[...]
    o1_ref[...] = jnp.dot(h, wr_ref[...], preferred_element_type=jnp.float32)
    o2_ref[...] = jnp.dot(h, w2_ref[...], preferred_element_type=jnp.float32)


def _hread(h, W_read, w2col):
    return pl.pallas_call(
        _hread_body,
        grid=(N // NBLK,),
        in_specs=[
            pl.BlockSpec((NBLK, H), lambda i: (i, 0)),
            pl.BlockSpec((H, H), lambda i: (0, 0)),
            pl.BlockSpec((H, 1), lambda i: (0, 0)),
        ],
        out_specs=[
            pl.BlockSpec((NBLK, H), lambda i: (i, 0)),
            pl.BlockSpec((NBLK, 1), lambda i: (i, 0)),
        ],
        out_shape=[
            jax.ShapeDtypeStruct((N, H), jnp.float32),
            jax.ShapeDtypeStruct((N, 1), jnp.float32),
        ],
    )(h, W_read, w2col)


def _seg0_body(gidr_ref, h_ref, o_ref):
    i = pl.program_id(0)
    oh_t = (gidr_ref[...] == lax.broadcasted_iota(jnp.int32, (G, NBLK), 0)
            ).astype(jnp.float32)
    part = jnp.dot(oh_t, h_ref[...], preferred_element_type=jnp.float32)

    @pl.when(i == 0)
    def _():
        o_ref[...] = jnp.zeros_like(o_ref)

    o_ref[...] += part


def _seg0(gids_row, h):
    return pl.pallas_call(
        _seg0_body,
        grid=(N // NBLK,),
        in_specs=[
            pl.BlockSpec((1, NBLK), lambda i: (0, i)),
            pl.BlockSpec((NBLK, H), lambda i: (i, 0)),
        ],
        out_specs=pl.BlockSpec((G, H), lambda i: (0, 0)),
        out_shape=jax.ShapeDtypeStruct((G, H), jnp.float32),
    )(gids_row, h)


def _tstep_body(g_ref, gidc_ref, gidr_ref, c2_ref, hwr_ref, w1_ref,
                zg_ref, msg_ref):
    i = pl.program_id(0)
    gw = jnp.dot(g_ref[...], w1_ref[...], preferred_element_type=jnp.float32)
    oh = (gidc_ref[...] == lax.broadcasted_iota(jnp.int32, (NBLK, G), 1)
          ).astype(jnp.float32)
    oh_t = (gidr_ref[...] == lax.broadcasted_iota(jnp.int32, (G, NBLK), 0)
            ).astype(jnp.float32)
    rl = jnp.dot(oh, gw, preferred_element_type=jnp.float32) + c2_ref[...]
    rl = jnp.where(rl >= 0.0, rl, 0.01 * rl)
    q = jnp.exp(rl)
    zg_part = jnp.dot(oh_t, q, preferred_element_type=jnp.float32)
    msg_part = jnp.dot(oh_t, q * hwr_ref[...], preferred_element_type=jnp.float32)

    @pl.when(i == 0)
    def _():
        zg_ref[...] = jnp.zeros_like(zg_ref)
        msg_ref[...] = jnp.zeros_like(msg_ref)

    zg_ref[...] += zg_part
    msg_ref[...] += msg_part


def _tstep(g, gids_col, gids_row, c2, hWr, w1col):
    return pl.pallas_call(
        _tstep_body,
        grid=(N // NBLK,),
        in_specs=[
            pl.BlockSpec((G, H), lambda i: (0, 0)),
            pl.BlockSpec((NBLK, 1), lambda i: (i, 0)),
            pl.BlockSpec((1, NBLK), lambda i: (0, i)),
            pl.BlockSpec((NBLK, 1), lambda i: (i, 0)),
            pl.BlockSpec((NBLK, H), lambda i: (i, 0)),
            pl.BlockSpec((H, 1), lambda i: (0, 0)),
        ],
        out_specs=[
            pl.BlockSpec((G, 1), lambda i: (0, 0)),
            pl.BlockSpec((G, H), lambda i: (0, 0)),
        ],
        out_shape=[
            jax.ShapeDtypeStruct((G, 1), jnp.float32),
            jax.ShapeDtypeStruct((G, H), jnp.float32),
        ],
    )(g, gids_col, gids_row, c2, hWr, w1col)


def _tgru_body(msg_ref, zg_ref, g_ref, *rest):
    wrefs = rest[:-1]
    o_ref = rest[-1]
    msg = msg_ref[...] * (1.0 / (zg_ref[...] + 1e-16))
    xi = jnp.where(msg > 0.0, msg, jnp.exp(msg) - 1.0)
    ws = tuple(w[...] for w in wrefs)
    o_ref[...] = _gru_math(xi, g_ref[...], ws)


def _tgru(msgp, zg, g, ws):
    wspecs = [pl.BlockSpec(w.shape, lambda i: tuple(0 for _ in w.shape)) for w in ws]
    return pl.pallas_call(
        _tgru_body,
        grid=(1,),
        in_specs=[
            pl.BlockSpec((G, H), lambda i: (0, 0)),
            pl.BlockSpec((G, 1), lambda i: (0, 0)),
            pl.BlockSpec((G, H), lambda i: (0, 0)),
        ] + wspecs,
        out_specs=pl.BlockSpec((G, H), lambda i: (0, 0)),
        out_shape=jax.ShapeDtypeStruct((G, H), jnp.float32),
    )(msgp, zg, g, *ws)


def _final_body(g_ref, W2_ref, charge_ref, out_ref):
    emb = jnp.dot(g_ref[...], W2_ref[...], preferred_element_type=jnp.float32)
    emb = jnp.concatenate([emb, jnp.zeros((MAXLEN - G, OUT), jnp.float32)], axis=0)
    col = lax.broadcasted_iota(jnp.int32, (MAXLEN, OUT), 1)
    row = lax.broadcasted_iota(jnp.int32, (MAXLEN, OUT), 0)
    is_last = row == MAXLEN - 1
    emb = jnp.where(is_last & (col == charge_ref[0]), 1.0, emb)
    emb = jnp.where(is_last & (col == OUT - 1), NCE, emb)
    out_ref[...] = emb


def _split3(Wih, Whh, bih, bhh):
    ws = [Wih[:, :H], Wih[:, H:2 * H], Wih[:, 2 * H:],
          Whh[:, :H], Whh[:, H:2 * H], Whh[:, 2 * H:],
          bih[:H].reshape(1, H), bih[H:2 * H].reshape(1, H), bih[2 * H:].reshape(1, H),
          bhh[:H].reshape(1, H), bhh[H:2 * H].reshape(1, H), bhh[2 * H:].reshape(1, H)]
    return tuple(ws)


def kernel(x, edge_index, edge_attr, graph_ids, charge, W1, We, att_e, att_d,
           Wm0, att_src, att_dst, Wm, gWih, gWhh, gbih, gbhh, w_read, W_read,
           mWih, mWhh, mbih, mbhh, W2):
    src = edge_index[0]
    dst = edge_index[1]
    zeros_n = jnp.zeros((N,), jnp.float32)
    zeros_rows = jnp.zeros((ZR, W40), jnp.float32)
    gws = _split3(gWih, gWhh, gbih, gbhh)
    mws = _split3(mWih, mWhh, mbih, mbhh)
    z36 = jnp.zeros((H, 1), jnp.float32)

    h = _mm(x, W1, act="leaky")

    def sparse_layer(h, Wtab, u, advec):
        table = _mm(h, Wtab)
        rows, _ = _sc_gather(table, src, advec)
        e, zp = _sc_edge(u.reshape(E), dst, advec, zeros_n)
        scaled = _scale_rows(rows, e.reshape(E, 1))
        agg = _sc_scatter(dst, scaled, zeros_rows)
        return _gru_node(agg, zp.T, h, gws), rows

    Wtab0 = jnp.concatenate([We[:H], att_d.reshape(H, 1), z36, z36, z36], axis=1)
    table0 = _mm(h, Wtab0)
    advec0 = table0[:, 36]
    rowsW, _ = _sc_gather(table0, src, advec0)
    u0 = _edge0(rowsW, edge_attr, We[H:], att_e.reshape(H, 1))
    Wm0t = jnp.concatenate([Wm0, z36, z36, z36, z36], axis=1)
    h, _ = sparse_layer(h, Wm0t, u0, advec0)

    for l in range(L - 1):
        Wtab = jnp.concatenate(
            [Wm[l], att_src[l].reshape(H, 1), att_dst[l].reshape(H, 1), z36, z36],
            axis=1)
        table = _mm(h, Wtab)
        advec = table[:, 37]
        rows, u = _sc_gather(table, src, table[:, 36])
        e, zp = _sc_edge(u, dst, advec, zeros_n)
        scaled = _scale_rows(rows, e.reshape(E, 1))
        agg = _sc_scatter(dst, scaled, zeros_rows)
        h = _gru_node(agg, zp.T, h, gws)

    gids_row = graph_ids.reshape(1, N)
    gids_col = graph_ids.reshape(N, 1)
    g = _seg0(gids_row, h)
    hWr, c2 = _hread(h, W_read, w_read[H:].reshape(H, 1))
    w1col = w_read[:H].reshape(H, 1)
    for t in range(T):
        zg, msgp = _tstep(g, gids_col, gids_row, c2, hWr, w1col)
        g = _tgru(msgp, zg, g, mws)

    charge_arr = jnp.asarray(charge, dtype=jnp.int32).reshape((1,))
    out = pl.pallas_call(
        _final_body,
        out_shape=jax.ShapeDtypeStruct((MAXLEN, OUT), jnp.float32),
    )(g, W2, charge_arr)
    return out
